# baseline (speedup 1.0000x reference)
import jax
import jax.numpy as jnp
from jax.experimental import pallas as pl

HEADS = 8
DH = 16
DOUT = 64


def _layer(x, src, dst, eattr, Wl, bl, Wr, br, We, att, bias, heads, outc):
    n = x.shape[0]
    xl = (x @ Wl + bl).reshape(n, heads, outc)
    xr = (x @ Wr + br).reshape(n, heads, outc)
    deg = jnp.zeros((n,), jnp.float32).at[dst].add(1.0)
    loop_attr = jax.ops.segment_sum(eattr, dst, num_segments=n) / jnp.maximum(deg, 1.0)[:, None]
    loops = jnp.arange(n, dtype=src.dtype)
    src2 = jnp.concatenate([src, loops])
    dst2 = jnp.concatenate([dst, loops])
    ea2 = jnp.concatenate([eattr, loop_attr], axis=0)
    ee = (ea2 @ We).reshape(-1, heads, outc)
    m = jax.nn.leaky_relu(xl[src2] + xr[dst2] + ee, 0.2)
    alpha = (m * att[None, :, :]).sum(-1)
    amax = jax.ops.segment_max(alpha, dst2, num_segments=n)
    alpha = jnp.exp(alpha - amax[dst2])
    den = jax.ops.segment_sum(alpha, dst2, num_segments=n)
    alpha = alpha / (den[dst2] + 1e-16)
    out = jax.ops.segment_sum(xl[src2] * alpha[..., None], dst2, num_segments=n)
    return out.reshape(n, heads * outc) + bias


def kernel(x, edge_index, edge_feats, Wl1, bl1, Wr1, br1, We1, att1, b1, Wl2, bl2, Wr2, br2, We2, att2, b2):
    src, dst = edge_index[0], edge_index[1]
    h = _layer(x, src, dst, edge_feats, Wl1, bl1, Wr1, br1, We1, att1, b1, HEADS, DH)
    h = jax.nn.elu(h)
    h = _layer(h, src, dst, edge_feats, Wl2, bl2, Wr2, br2, We2, att2, b2, 1, DOUT)
    return h, jax.nn.log_softmax(h, axis=1)


# speedup vs baseline: 16.5166x; 16.5166x over previous
import functools

import jax
import jax.numpy as jnp
from jax import lax
from jax.experimental import pallas as pl
from jax.experimental.pallas import tpu as pltpu
from jax.experimental.pallas import tpu_sc as plsc

N = 10000
E = 320000
DIN = 128
HEADS = 8
DH = 16
D1 = HEADS * DH
DOUT = 64

NC, NS = 2, 16
NW = NC * NS
TPE = E // NW
B = 40
NB = TPE // B
CCH = 80
NCHUNK = N // CCH
CPS = -(-NCHUNK // NS)

_f32 = jnp.float32


def _mm_ee_body(ef, We1, We2, o1, o2):
    ef_ = ef[...]
    o1[...] = jnp.dot(ef_, We1[...], preferred_element_type=_f32)
    o2[:, 0:64] = jnp.dot(ef_, We2[...], preferred_element_type=_f32)
    o2[:, 64:65] = jnp.ones((ef_.shape[0], 1), _f32)
    o2[:, 65:80] = jnp.zeros((ef_.shape[0], 15), _f32)


def _mm_ee(edge_feats, We1, We2):
    RB = 2000
    return pl.pallas_call(
        _mm_ee_body,
        grid=(E // RB,),
        in_specs=[
            pl.BlockSpec((RB, DIN), lambda i: (i, 0)),
            pl.BlockSpec((DIN, D1), lambda i: (0, 0)),
            pl.BlockSpec((DIN, DOUT), lambda i: (0, 0)),
        ],
        out_specs=[
            pl.BlockSpec((RB, D1), lambda i: (i, 0)),
            pl.BlockSpec((RB, 80), lambda i: (i, 0)),
        ],
        out_shape=[
            jax.ShapeDtypeStruct((E, D1), _f32),
            jax.ShapeDtypeStruct((E, 80), _f32),
        ],
    )(edge_feats, We1, We2)


def _mm_x1_body(x, Wl, bl, Wr, br, ol, orr):
    x_ = x[...]
    ol[...] = jnp.dot(x_, Wl[...], preferred_element_type=_f32) + bl[...]
    orr[...] = jnp.dot(x_, Wr[...], preferred_element_type=_f32) + br[...]


def _mm_x1(x, Wl1, bl1, Wr1, br1):
    RB = 1000
    return pl.pallas_call(
        _mm_x1_body,
        grid=(N // RB,),
        in_specs=[
            pl.BlockSpec((RB, DIN), lambda i: (i, 0)),
            pl.BlockSpec((DIN, D1), lambda i: (0, 0)),
            pl.BlockSpec((1, D1), lambda i: (0, 0)),
            pl.BlockSpec((DIN, D1), lambda i: (0, 0)),
            pl.BlockSpec((1, D1), lambda i: (0, 0)),
        ],
        out_specs=[
            pl.BlockSpec((RB, D1), lambda i: (i, 0)),
            pl.BlockSpec((RB, D1), lambda i: (i, 0)),
        ],
        out_shape=[
            jax.ShapeDtypeStruct((N, D1), _f32),
            jax.ShapeDtypeStruct((N, D1), _f32),
        ],
    )(x, Wl1, bl1, Wr1, br1)


def _f1_body(s1a, s1b, a1a, a1b, a2a, a2b, xl, xr, attf, G, G2, b1,
             Wl2, bl2, Wr2, br2, oxl2, oxr2):
    num = s1a[:, 0:D1] + s1b[:, 0:D1]
    den8 = s1a[:, D1:D1 + HEADS] + s1b[:, D1:D1 + HEADS]
    es1 = a1a[...] + a1b[...]
    deg = a2a[:, 64:65] + a2b[:, 64:65]
    loop_ee = es1 / jnp.maximum(deg, 1.0)
    xl_ = xl[...]
    t = xl_ + xr[...] + loop_ee
    m = jnp.maximum(t, 0.0) + 0.2 * jnp.minimum(t, 0.0)
    a8 = jnp.dot(m * attf[...], G[...], preferred_element_type=_f32)
    p8 = jnp.exp(a8)
    dent = den8 + p8
    numt = num + xl_ * jnp.dot(p8, G2[...], preferred_element_type=_f32)
    out = numt / (jnp.dot(dent, G2[...], preferred_element_type=_f32) + 1e-16)
    out = out + b1[...]
    h = jnp.where(out > 0.0, out, jnp.exp(jnp.minimum(out, 0.0)) - 1.0)
    oxl2[...] = jnp.dot(h, Wl2[...], preferred_element_type=_f32) + bl2[...]
    oxr2[...] = jnp.dot(h, Wr2[...], preferred_element_type=_f32) + br2[...]


def _f1(S1, A1, A2, xl1, xr1, attf1, G, G2, b1, Wl2, bl2, Wr2, br2):
    RB = 1000
    nb = N // RB
    return pl.pallas_call(
        _f1_body,
        grid=(nb,),
        in_specs=[
            pl.BlockSpec((RB, D1 + 16), lambda i: (i, 0)),
            pl.BlockSpec((RB, D1 + 16), lambda i: (i + nb, 0)),
            pl.BlockSpec((RB, D1), lambda i: (i, 0)),
            pl.BlockSpec((RB, D1), lambda i: (i + nb, 0)),
            pl.BlockSpec((RB, 80), lambda i: (i, 0)),
            pl.BlockSpec((RB, 80), lambda i: (i + nb, 0)),
            pl.BlockSpec((RB, D1), lambda i: (i, 0)),
            pl.BlockSpec((RB, D1), lambda i: (i, 0)),
            pl.BlockSpec((1, D1), lambda i: (0, 0)),
            pl.BlockSpec((D1, HEADS), lambda i: (0, 0)),
            pl.BlockSpec((HEADS, D1), lambda i: (0, 0)),
            pl.BlockSpec((1, D1), lambda i: (0, 0)),
            pl.BlockSpec((D1, DOUT), lambda i: (0, 0)),
            pl.BlockSpec((1, DOUT), lambda i: (0, 0)),
            pl.BlockSpec((D1, DOUT), lambda i: (0, 0)),
            pl.BlockSpec((1, DOUT), lambda i: (0, 0)),
        ],
        out_specs=[
            pl.BlockSpec((RB, DOUT), lambda i: (i, 0)),
            pl.BlockSpec((RB, DOUT), lambda i: (i, 0)),
        ],
        out_shape=[
            jax.ShapeDtypeStruct((N, DOUT), _f32),
            jax.ShapeDtypeStruct((N, DOUT), _f32),
        ],
    )(S1, S1, A1, A1, A2, A2, xl1, xr1, attf1, G, G2, b1, Wl2, bl2, Wr2, br2)


def _f2_body(s2a, s2b, a2a, a2b, xl, xr, attf, b2, oo, ol):
    num = s2a[:, 0:DOUT] + s2b[:, 0:DOUT]
    den = s2a[:, DOUT:DOUT + 1] + s2b[:, DOUT:DOUT + 1]
    es2 = a2a[:, 0:DOUT] + a2b[:, 0:DOUT]
    deg = a2a[:, 64:65] + a2b[:, 64:65]
    loop_ee = es2 / jnp.maximum(deg, 1.0)
    xl_ = xl[...]
    t = xl_ + xr[...] + loop_ee
    m = jnp.maximum(t, 0.0) + 0.2 * jnp.minimum(t, 0.0)
    a = jnp.sum(m * attf[...], axis=1, keepdims=True)
    p = jnp.exp(a)
    dent = den + p
    numt = num + xl_ * p
    o = numt / (dent + 1e-16) + b2[...]
    mx = jnp.max(o, axis=1, keepdims=True)
    sh = o - mx
    lse = jnp.log(jnp.sum(jnp.exp(sh), axis=1, keepdims=True))
    oo[...] = o
    ol[...] = sh - lse


def _f2(S2, A2, xl2, xr2, att2f, b2):
    RB = 1000
    nb = N // RB
    return pl.pallas_call(
        _f2_body,
        grid=(nb,),
        in_specs=[
            pl.BlockSpec((RB, 80), lambda i: (i, 0)),
            pl.BlockSpec((RB, 80), lambda i: (i + nb, 0)),
            pl.BlockSpec((RB, 80), lambda i: (i, 0)),
            pl.BlockSpec((RB, 80), lambda i: (i + nb, 0)),
            pl.BlockSpec((RB, DOUT), lambda i: (i, 0)),
            pl.BlockSpec((RB, DOUT), lambda i: (i, 0)),
            pl.BlockSpec((1, DOUT), lambda i: (0, 0)),
            pl.BlockSpec((1, DOUT), lambda i: (0, 0)),
        ],
        out_specs=[
            pl.BlockSpec((RB, DOUT), lambda i: (i, 0)),
            pl.BlockSpec((RB, DOUT), lambda i: (i, 0)),
        ],
        out_shape=[
            jax.ShapeDtypeStruct((N, DOUT), _f32),
            jax.ShapeDtypeStruct((N, DOUT), _f32),
        ],
    )(S2, S2, A2, A2, xl2, xr2, att2f, b2)


_MESH = dict(core_axis_name="c", subcore_axis_name="s", num_cores=NC,
             num_subcores=NS)


def _sc_scatter(EW):
    def body(dst_h, ee_h, z_h, o_h, dstv, ev, cbuf, es, sem):
        c = lax.axis_index("c")
        s = lax.axis_index("s")
        wid = c * NS + s
        pltpu.sync_copy(z_h, cbuf)
        for k in range(CPS):
            idx = s + k * NS

            @pl.when(idx < NCHUNK)
            def _():
                pltpu.sync_copy(cbuf, es.at[pl.ds(idx * CCH, CCH)])
        plsc.subcore_barrier()

        def batch(i, carry):
            base = wid * TPE + i * B
            pltpu.sync_copy(dst_h.at[pl.ds(base, B)], dstv)
            pltpu.sync_copy(ee_h.at[pl.ds(base, B)], ev)
            pltpu.sync_copy(ev, es.at[dstv], add=True)
            return carry

        lax.fori_loop(0, NB, batch, 0)
        plsc.subcore_barrier()
        for k in range(CPS):
            idx = s + k * NS

            @pl.when(idx < NCHUNK)
            def _():
                r0 = idx * CCH
                pltpu.sync_copy(es.at[pl.ds(r0, CCH)], cbuf)
                pltpu.sync_copy(cbuf, o_h.at[pl.ds(c * N + r0, CCH)])

    return pl.kernel(
        body,
        out_type=jax.ShapeDtypeStruct((NC * N, EW), _f32),
        mesh=plsc.VectorSubcoreMesh(**_MESH),
        scratch_types=[
            pltpu.VMEM((B,), jnp.int32),
            pltpu.VMEM((B, EW), _f32),
            pltpu.VMEM((CCH, EW), _f32),
            pltpu.VMEM_SHARED((N, EW), _f32),
            pltpu.SemaphoreType.DMA,
        ],
        compiler_params=pltpu.CompilerParams(use_tc_tiling_on_sc=False, needs_layout_passes=False),
    )


def _sc_edge(D, H, dh):
    W = D + 16

    def body(src_h, dst_h, xl_h, xr_h, ee_h, att_h, z_h, S_h,
             srcv, dstv, xlv, xrv, eev, nxv, attv, cbuf, acc, sem):
        c = lax.axis_index("c")
        s = lax.axis_index("s")
        wid = c * NS + s
        pltpu.sync_copy(att_h, attv)
        pltpu.sync_copy(z_h, cbuf)
        for k in range(CPS):
            idx = s + k * NS

            @pl.when(idx < NCHUNK)
            def _():
                pltpu.sync_copy(cbuf, acc.at[pl.ds(idx * CCH, CCH)])
        plsc.subcore_barrier()

        lane = lax.iota(jnp.int32, 16)

        def edge(e, carry):
            den_acc = jnp.full((16,), -1e30, _f32)
            for h in range(H):
                a = jnp.float32(0.0)
                for kk in range(dh // 16):
                    sl = pl.ds(h * dh + kk * 16, 16)
                    xlvv = xlv[e, sl]
                    t = xlvv + xrv[e, sl] + eev[e, sl]
                    m = jnp.maximum(t, 0.0) + 0.2 * jnp.minimum(t, 0.0)
                    a = a + jnp.sum(m * attv[h, pl.ds(kk * 16, 16)])
                    nxv[e, sl] = xlvv
                den_acc = jnp.where(lane == h, a, den_acc)
            pb = jnp.exp(den_acc)
            nxv[e, pl.ds(D, 16)] = pb
            for h in range(H):
                p = pb[h]
                for kk in range(dh // 16):
                    sl = pl.ds(h * dh + kk * 16, 16)
                    nxv[e, sl] = p * nxv[e, sl]
            return carry

        def batch(i, carry):
            base = wid * TPE + i * B
            pltpu.sync_copy(src_h.at[pl.ds(base, B)], srcv)
            pltpu.sync_copy(dst_h.at[pl.ds(base, B)], dstv)
            d1 = pltpu.async_copy(xl_h.at[srcv], xlv, sem)
            d2 = pltpu.async_copy(xr_h.at[dstv], xrv, sem)
            d3 = pltpu.async_copy(ee_h.at[pl.ds(base, B)], eev, sem)
            d1.wait()
            d2.wait()
            d3.wait()
            lax.fori_loop(0, B, edge, 0)
            pltpu.sync_copy(nxv, acc.at[dstv], add=True)
            return carry

        lax.fori_loop(0, NB, batch, 0)
        plsc.subcore_barrier()
        for k in range(CPS):
            idx = s + k * NS

            @pl.when(idx < NCHUNK)
            def _():
                r0 = idx * CCH
                pltpu.sync_copy(acc.at[pl.ds(r0, CCH)], cbuf)
                pltpu.sync_copy(cbuf, S_h.at[pl.ds(c * N + r0, CCH)])

    EW = 80 if D == DOUT else D
    return pl.kernel(
        body,
        out_type=jax.ShapeDtypeStruct((NC * N, W), _f32),
        mesh=plsc.VectorSubcoreMesh(**_MESH),
        scratch_types=[
            pltpu.VMEM((B,), jnp.int32),
            pltpu.VMEM((B,), jnp.int32),
            pltpu.VMEM((B, D), _f32),
            pltpu.VMEM((B, D), _f32),
            pltpu.VMEM((B, EW), _f32),
            pltpu.VMEM((B, W), _f32),
            pltpu.VMEM((H, dh), _f32),
            pltpu.VMEM((CCH, W), _f32),
            pltpu.VMEM_SHARED((N, W), _f32),
            pltpu.SemaphoreType.DMA,
        ],
        compiler_params=pltpu.CompilerParams(use_tc_tiling_on_sc=False, needs_layout_passes=False),
    )


def kernel(x, edge_index, edge_feats, Wl1, bl1, Wr1, br1, We1, att1, b1,
           Wl2, bl2, Wr2, br2, We2, att2, b2):
    src = edge_index[0]
    dst = edge_index[1]

    ee1, ee2x = _mm_ee(edge_feats, We1, We2)
    xl1, xr1 = _mm_x1(x, Wl1, bl1.reshape(1, D1), Wr1, br1.reshape(1, D1))

    z128 = jnp.zeros((CCH, D1), _f32)
    z80 = jnp.zeros((CCH, 80), _f32)
    z144 = jnp.zeros((CCH, D1 + 16), _f32)

    A1 = _sc_scatter(D1)(dst, ee1, z128)
    A2 = _sc_scatter(80)(dst, ee2x, z80)
    S1 = _sc_edge(D1, HEADS, DH)(src, dst, xl1, xr1, ee1, att1, z144)

    attf1 = att1.reshape(1, D1)
    G = jnp.repeat(jnp.eye(HEADS, dtype=_f32), DH, axis=0)
    G2 = G.T
    xl2, xr2 = _f1(S1, A1, A2, xl1, xr1, attf1, G, G2, b1.reshape(1, D1),
                   Wl2, bl2.reshape(1, DOUT), Wr2, br2.reshape(1, DOUT))

    S2 = _sc_edge(DOUT, 1, DOUT)(src, dst, xl2, xr2, ee2x, att2, z80)
    o, lsm = _f2(S2, A2, xl2, xr2, att2.reshape(1, DOUT), b2.reshape(1, DOUT))
    return o, lsm


# speedup vs baseline: 19.2353x; 1.1646x over previous
import functools

import jax
import jax.numpy as jnp
from jax import lax
from jax.experimental import pallas as pl
from jax.experimental.pallas import tpu as pltpu
from jax.experimental.pallas import tpu_sc as plsc

N = 10000
E = 320000
DIN = 128
HEADS = 8
DH = 16
D1 = HEADS * DH
DOUT = 64

NC, NS = 2, 16
NW = NC * NS
TPE = E // NW
B = 40
NB = TPE // B
CCH = 80
NCHUNK = N // CCH
CPS = -(-NCHUNK // NS)

_f32 = jnp.float32


def _mm_ee_body(ef, We1, We2, o1, o2):
    ef_ = ef[...]
    o1[...] = jnp.dot(ef_, We1[...], preferred_element_type=_f32)
    o2[:, 0:64] = jnp.dot(ef_, We2[...], preferred_element_type=_f32)
    o2[:, 64:65] = jnp.ones((ef_.shape[0], 1), _f32)
    o2[:, 65:80] = jnp.zeros((ef_.shape[0], 15), _f32)


def _mm_ee(edge_feats, We1, We2):
    RB = 2000
    return pl.pallas_call(
        _mm_ee_body,
        grid=(E // RB,),
        in_specs=[
            pl.BlockSpec((RB, DIN), lambda i: (i, 0)),
            pl.BlockSpec((DIN, D1), lambda i: (0, 0)),
            pl.BlockSpec((DIN, DOUT), lambda i: (0, 0)),
        ],
        out_specs=[
            pl.BlockSpec((RB, D1), lambda i: (i, 0)),
            pl.BlockSpec((RB, 80), lambda i: (i, 0)),
        ],
        out_shape=[
            jax.ShapeDtypeStruct((E, D1), _f32),
            jax.ShapeDtypeStruct((E, 80), _f32),
        ],
    )(edge_feats, We1, We2)


def _mm_x1_body(x, Wl, bl, Wr, br, ol, orr):
    x_ = x[...]
    ol[...] = jnp.dot(x_, Wl[...], preferred_element_type=_f32) + bl[...]
    orr[...] = jnp.dot(x_, Wr[...], preferred_element_type=_f32) + br[...]


def _mm_x1(x, Wl1, bl1, Wr1, br1):
    RB = 1000
    return pl.pallas_call(
        _mm_x1_body,
        grid=(N // RB,),
        in_specs=[
            pl.BlockSpec((RB, DIN), lambda i: (i, 0)),
            pl.BlockSpec((DIN, D1), lambda i: (0, 0)),
            pl.BlockSpec((1, D1), lambda i: (0, 0)),
            pl.BlockSpec((DIN, D1), lambda i: (0, 0)),
            pl.BlockSpec((1, D1), lambda i: (0, 0)),
        ],
        out_specs=[
            pl.BlockSpec((RB, D1), lambda i: (i, 0)),
            pl.BlockSpec((RB, D1), lambda i: (i, 0)),
        ],
        out_shape=[
            jax.ShapeDtypeStruct((N, D1), _f32),
            jax.ShapeDtypeStruct((N, D1), _f32),
        ],
    )(x, Wl1, bl1, Wr1, br1)


def _f1_body(s1a, s1b, a1a, a1b, a2a, a2b, xl, xr, attf, G, G2, b1,
             Wl2, bl2, Wr2, br2, oxl2, oxr2):
    num = s1a[:, 0:D1] + s1b[:, 0:D1]
    den8 = s1a[:, D1:D1 + HEADS] + s1b[:, D1:D1 + HEADS]
    es1 = a1a[...] + a1b[...]
    deg = a2a[:, 64:65] + a2b[:, 64:65]
    loop_ee = es1 / jnp.maximum(deg, 1.0)
    xl_ = xl[...]
    t = xl_ + xr[...] + loop_ee
    m = jnp.maximum(t, 0.0) + 0.2 * jnp.minimum(t, 0.0)
    a8 = jnp.dot(m * attf[...], G[...], preferred_element_type=_f32)
    p8 = jnp.exp(a8)
    dent = den8 + p8
    numt = num + xl_ * jnp.dot(p8, G2[...], preferred_element_type=_f32)
    out = numt / (jnp.dot(dent, G2[...], preferred_element_type=_f32) + 1e-16)
    out = out + b1[...]
    h = jnp.where(out > 0.0, out, jnp.exp(jnp.minimum(out, 0.0)) - 1.0)
    oxl2[...] = jnp.dot(h, Wl2[...], preferred_element_type=_f32) + bl2[...]
    oxr2[...] = jnp.dot(h, Wr2[...], preferred_element_type=_f32) + br2[...]


def _f1(S1, A1, A2, xl1, xr1, attf1, G, G2, b1, Wl2, bl2, Wr2, br2):
    RB = 1000
    nb = N // RB
    return pl.pallas_call(
        _f1_body,
        grid=(nb,),
        in_specs=[
            pl.BlockSpec((RB, D1 + 16), lambda i: (i, 0)),
            pl.BlockSpec((RB, D1 + 16), lambda i: (i + nb, 0)),
            pl.BlockSpec((RB, D1), lambda i: (i, 0)),
            pl.BlockSpec((RB, D1), lambda i: (i + nb, 0)),
            pl.BlockSpec((RB, 80), lambda i: (i, 0)),
            pl.BlockSpec((RB, 80), lambda i: (i + nb, 0)),
            pl.BlockSpec((RB, D1), lambda i: (i, 0)),
            pl.BlockSpec((RB, D1), lambda i: (i, 0)),
            pl.BlockSpec((1, D1), lambda i: (0, 0)),
            pl.BlockSpec((D1, HEADS), lambda i: (0, 0)),
            pl.BlockSpec((HEADS, D1), lambda i: (0, 0)),
            pl.BlockSpec((1, D1), lambda i: (0, 0)),
            pl.BlockSpec((D1, DOUT), lambda i: (0, 0)),
            pl.BlockSpec((1, DOUT), lambda i: (0, 0)),
            pl.BlockSpec((D1, DOUT), lambda i: (0, 0)),
            pl.BlockSpec((1, DOUT), lambda i: (0, 0)),
        ],
        out_specs=[
            pl.BlockSpec((RB, DOUT), lambda i: (i, 0)),
            pl.BlockSpec((RB, DOUT), lambda i: (i, 0)),
        ],
        out_shape=[
            jax.ShapeDtypeStruct((N, DOUT), _f32),
            jax.ShapeDtypeStruct((N, DOUT), _f32),
        ],
    )(S1, S1, A1, A1, A2, A2, xl1, xr1, attf1, G, G2, b1, Wl2, bl2, Wr2, br2)


def _f2_body(s2a, s2b, a2a, a2b, xl, xr, attf, b2, oo, ol):
    num = s2a[:, 0:DOUT] + s2b[:, 0:DOUT]
    den = s2a[:, DOUT:DOUT + 1] + s2b[:, DOUT:DOUT + 1]
    es2 = a2a[:, 0:DOUT] + a2b[:, 0:DOUT]
    deg = a2a[:, 64:65] + a2b[:, 64:65]
    loop_ee = es2 / jnp.maximum(deg, 1.0)
    xl_ = xl[...]
    t = xl_ + xr[...] + loop_ee
    m = jnp.maximum(t, 0.0) + 0.2 * jnp.minimum(t, 0.0)
    a = jnp.sum(m * attf[...], axis=1, keepdims=True)
    p = jnp.exp(a)
    dent = den + p
    numt = num + xl_ * p
    o = numt / (dent + 1e-16) + b2[...]
    mx = jnp.max(o, axis=1, keepdims=True)
    sh = o - mx
    lse = jnp.log(jnp.sum(jnp.exp(sh), axis=1, keepdims=True))
    oo[...] = o
    ol[...] = sh - lse


def _f2(S2, A2, xl2, xr2, att2f, b2):
    RB = 1000
    nb = N // RB
    return pl.pallas_call(
        _f2_body,
        grid=(nb,),
        in_specs=[
            pl.BlockSpec((RB, 80), lambda i: (i, 0)),
            pl.BlockSpec((RB, 80), lambda i: (i + nb, 0)),
            pl.BlockSpec((RB, 80), lambda i: (i, 0)),
            pl.BlockSpec((RB, 80), lambda i: (i + nb, 0)),
            pl.BlockSpec((RB, DOUT), lambda i: (i, 0)),
            pl.BlockSpec((RB, DOUT), lambda i: (i, 0)),
            pl.BlockSpec((1, DOUT), lambda i: (0, 0)),
            pl.BlockSpec((1, DOUT), lambda i: (0, 0)),
        ],
        out_specs=[
            pl.BlockSpec((RB, DOUT), lambda i: (i, 0)),
            pl.BlockSpec((RB, DOUT), lambda i: (i, 0)),
        ],
        out_shape=[
            jax.ShapeDtypeStruct((N, DOUT), _f32),
            jax.ShapeDtypeStruct((N, DOUT), _f32),
        ],
    )(S2, S2, A2, A2, xl2, xr2, att2f, b2)


_MESH = dict(core_axis_name="c", subcore_axis_name="s", num_cores=NC,
             num_subcores=NS)


def _sc_scatter(EW):
    def body(dst_h, ee_h, z_h, o_h, dstv, ev, cbuf, es, sem):
        c = lax.axis_index("c")
        s = lax.axis_index("s")
        wid = c * NS + s
        pltpu.sync_copy(z_h, cbuf)
        for k in range(CPS):
            idx = s + k * NS

            @pl.when(idx < NCHUNK)
            def _():
                pltpu.sync_copy(cbuf, es.at[pl.ds(idx * CCH, CCH)])
        plsc.subcore_barrier()

        def batch(i, carry):
            base = wid * TPE + i * B
            pltpu.sync_copy(dst_h.at[pl.ds(base, B)], dstv)
            pltpu.sync_copy(ee_h.at[pl.ds(base, B)], ev)
            pltpu.sync_copy(ev, es.at[dstv], add=True)
            return carry

        lax.fori_loop(0, NB, batch, 0)
        plsc.subcore_barrier()
        for k in range(CPS):
            idx = s + k * NS

            @pl.when(idx < NCHUNK)
            def _():
                r0 = idx * CCH
                pltpu.sync_copy(es.at[pl.ds(r0, CCH)], cbuf)
                pltpu.sync_copy(cbuf, o_h.at[pl.ds(c * N + r0, CCH)])

    return pl.kernel(
        body,
        out_type=jax.ShapeDtypeStruct((NC * N, EW), _f32),
        mesh=plsc.VectorSubcoreMesh(**_MESH),
        scratch_types=[
            pltpu.VMEM((B,), jnp.int32),
            pltpu.VMEM((B, EW), _f32),
            pltpu.VMEM((CCH, EW), _f32),
            pltpu.VMEM_SHARED((N, EW), _f32),
            pltpu.SemaphoreType.DMA,
        ],
        compiler_params=pltpu.CompilerParams(use_tc_tiling_on_sc=False, needs_layout_passes=False),
    )


def _sc_edge(D, H, dh):
    W = D + 16

    def body(src_h, dst_h, xl_h, xr_h, ee_h, att_h, z_h, S_h,
             srcv, dstv, xlv, xrv, eev, nxv, attv, cbuf, acc, sem):
        c = lax.axis_index("c")
        s = lax.axis_index("s")
        wid = c * NS + s
        pltpu.sync_copy(att_h, attv)
        pltpu.sync_copy(z_h, cbuf)
        for k in range(CPS):
            idx = s + k * NS

            @pl.when(idx < NCHUNK)
            def _():
                pltpu.sync_copy(cbuf, acc.at[pl.ds(idx * CCH, CCH)])
        plsc.subcore_barrier()

        lane = lax.iota(jnp.int32, 16)

        def edge(e):
            den_acc = jnp.full((16,), -1e30, _f32)
            for h in range(H):
                a = jnp.float32(0.0)
                for kk in range(dh // 16):
                    sl = pl.ds(h * dh + kk * 16, 16)
                    xlvv = xlv[e, sl]
                    t = xlvv + xrv[e, sl] + eev[e, sl]
                    m = jnp.maximum(t, 0.0) + 0.2 * jnp.minimum(t, 0.0)
                    a = a + jnp.sum(m * attv[h, pl.ds(kk * 16, 16)])
                    nxv[e, sl] = xlvv
                den_acc = jnp.where(lane == h, a, den_acc)
            pb = jnp.exp(den_acc)
            nxv[e, pl.ds(D, 16)] = pb
            for h in range(H):
                p = pb[h]
                for kk in range(dh // 16):
                    sl = pl.ds(h * dh + kk * 16, 16)
                    nxv[e, sl] = p * nxv[e, sl]

        def batch(i, carry):
            base = wid * TPE + i * B
            pltpu.sync_copy(src_h.at[pl.ds(base, B)], srcv)
            pltpu.sync_copy(dst_h.at[pl.ds(base, B)], dstv)
            d1 = pltpu.async_copy(xl_h.at[srcv], xlv, sem)
            d2 = pltpu.async_copy(xr_h.at[dstv], xrv, sem)
            d3 = pltpu.async_copy(ee_h.at[pl.ds(base, B)], eev, sem)
            d1.wait()
            d2.wait()
            d3.wait()
            for e in range(B):
                edge(e)
            pltpu.sync_copy(nxv, acc.at[dstv], add=True)
            return carry

        lax.fori_loop(0, NB, batch, 0)
        plsc.subcore_barrier()
        for k in range(CPS):
            idx = s + k * NS

            @pl.when(idx < NCHUNK)
            def _():
                r0 = idx * CCH
                pltpu.sync_copy(acc.at[pl.ds(r0, CCH)], cbuf)
                pltpu.sync_copy(cbuf, S_h.at[pl.ds(c * N + r0, CCH)])

    EW = 80 if D == DOUT else D
    return pl.kernel(
        body,
        out_type=jax.ShapeDtypeStruct((NC * N, W), _f32),
        mesh=plsc.VectorSubcoreMesh(**_MESH),
        scratch_types=[
            pltpu.VMEM((B,), jnp.int32),
            pltpu.VMEM((B,), jnp.int32),
            pltpu.VMEM((B, D), _f32),
            pltpu.VMEM((B, D), _f32),
            pltpu.VMEM((B, EW), _f32),
            pltpu.VMEM((B, W), _f32),
            pltpu.VMEM((H, dh), _f32),
            pltpu.VMEM((CCH, W), _f32),
            pltpu.VMEM_SHARED((N, W), _f32),
            pltpu.SemaphoreType.DMA,
        ],
        compiler_params=pltpu.CompilerParams(use_tc_tiling_on_sc=False, needs_layout_passes=False),
    )


def kernel(x, edge_index, edge_feats, Wl1, bl1, Wr1, br1, We1, att1, b1,
           Wl2, bl2, Wr2, br2, We2, att2, b2):
    src = edge_index[0]
    dst = edge_index[1]

    ee1, ee2x = _mm_ee(edge_feats, We1, We2)
    xl1, xr1 = _mm_x1(x, Wl1, bl1.reshape(1, D1), Wr1, br1.reshape(1, D1))

    z128 = jnp.zeros((CCH, D1), _f32)
    z80 = jnp.zeros((CCH, 80), _f32)
    z144 = jnp.zeros((CCH, D1 + 16), _f32)

    A1 = _sc_scatter(D1)(dst, ee1, z128)
    A2 = _sc_scatter(80)(dst, ee2x, z80)
    S1 = _sc_edge(D1, HEADS, DH)(src, dst, xl1, xr1, ee1, att1, z144)

    attf1 = att1.reshape(1, D1)
    G = jnp.repeat(jnp.eye(HEADS, dtype=_f32), DH, axis=0)
    G2 = G.T
    xl2, xr2 = _f1(S1, A1, A2, xl1, xr1, attf1, G, G2, b1.reshape(1, D1),
                   Wl2, bl2.reshape(1, DOUT), Wr2, br2.reshape(1, DOUT))

    S2 = _sc_edge(DOUT, 1, DOUT)(src, dst, xl2, xr2, ee2x, att2, z80)
    o, lsm = _f2(S2, A2, xl2, xr2, att2.reshape(1, DOUT), b2.reshape(1, DOUT))
    return o, lsm


# speedup vs baseline: 29.3693x; 1.5268x over previous
import functools

import jax
import jax.numpy as jnp
from jax import lax
from jax.experimental import pallas as pl
from jax.experimental.pallas import tpu as pltpu
from jax.experimental.pallas import tpu_sc as plsc

N = 10000
E = 320000
DIN = 128
HEADS = 8
DH = 16
D1 = HEADS * DH
DOUT = 64

NC, NS = 2, 16
NW = NC * NS
TPE = E // NW
B = 40
NB = TPE // B
CCH = 80
NCHUNK = N // CCH
CPS = -(-NCHUNK // NS)

_f32 = jnp.float32


def _mm_ee_body(ef, We1, We2, o1, o2):
    ef_ = ef[...]
    o1[...] = jnp.dot(ef_, We1[...], preferred_element_type=_f32)
    o2[:, 0:64] = jnp.dot(ef_, We2[...], preferred_element_type=_f32)
    o2[:, 64:65] = jnp.ones((ef_.shape[0], 1), _f32)
    o2[:, 65:80] = jnp.zeros((ef_.shape[0], 15), _f32)


def _mm_ee(edge_feats, We1, We2):
    RB = 2000
    return pl.pallas_call(
        _mm_ee_body,
        grid=(E // RB,),
        in_specs=[
            pl.BlockSpec((RB, DIN), lambda i: (i, 0)),
            pl.BlockSpec((DIN, D1), lambda i: (0, 0)),
            pl.BlockSpec((DIN, DOUT), lambda i: (0, 0)),
        ],
        out_specs=[
            pl.BlockSpec((RB, D1), lambda i: (i, 0)),
            pl.BlockSpec((RB, 80), lambda i: (i, 0)),
        ],
        out_shape=[
            jax.ShapeDtypeStruct((E, D1), _f32),
            jax.ShapeDtypeStruct((E, 80), _f32),
        ],
    )(edge_feats, We1, We2)


def _mm_x1_body(x, Wl, bl, Wr, br, ol, orr):
    x_ = x[...]
    ol[...] = jnp.dot(x_, Wl[...], preferred_element_type=_f32) + bl[...]
    orr[...] = jnp.dot(x_, Wr[...], preferred_element_type=_f32) + br[...]


def _mm_x1(x, Wl1, bl1, Wr1, br1):
    RB = 1000
    return pl.pallas_call(
        _mm_x1_body,
        grid=(N // RB,),
        in_specs=[
            pl.BlockSpec((RB, DIN), lambda i: (i, 0)),
            pl.BlockSpec((DIN, D1), lambda i: (0, 0)),
            pl.BlockSpec((1, D1), lambda i: (0, 0)),
            pl.BlockSpec((DIN, D1), lambda i: (0, 0)),
            pl.BlockSpec((1, D1), lambda i: (0, 0)),
        ],
        out_specs=[
            pl.BlockSpec((RB, D1), lambda i: (i, 0)),
            pl.BlockSpec((RB, D1), lambda i: (i, 0)),
        ],
        out_shape=[
            jax.ShapeDtypeStruct((N, D1), _f32),
            jax.ShapeDtypeStruct((N, D1), _f32),
        ],
    )(x, Wl1, bl1, Wr1, br1)


def _f1_body(s1a, s1b, a1a, a1b, a2a, a2b, xl, xr, attf, G, G2, b1,
             Wl2, bl2, Wr2, br2, oxl2, oxr2):
    num = s1a[:, 0:D1] + s1b[:, 0:D1]
    den8 = s1a[:, D1:D1 + HEADS] + s1b[:, D1:D1 + HEADS]
    es1 = a1a[...] + a1b[...]
    deg = a2a[:, 64:65] + a2b[:, 64:65]
    loop_ee = es1 / jnp.maximum(deg, 1.0)
    xl_ = xl[...]
    t = xl_ + xr[...] + loop_ee
    m = jnp.maximum(t, 0.0) + 0.2 * jnp.minimum(t, 0.0)
    a8 = jnp.dot(m * attf[...], G[...], preferred_element_type=_f32)
    p8 = jnp.exp(a8)
    dent = den8 + p8
    numt = num + xl_ * jnp.dot(p8, G2[...], preferred_element_type=_f32)
    out = numt / (jnp.dot(dent, G2[...], preferred_element_type=_f32) + 1e-16)
    out = out + b1[...]
    h = jnp.where(out > 0.0, out, jnp.exp(jnp.minimum(out, 0.0)) - 1.0)
    oxl2[...] = jnp.dot(h, Wl2[...], preferred_element_type=_f32) + bl2[...]
    oxr2[...] = jnp.dot(h, Wr2[...], preferred_element_type=_f32) + br2[...]


def _f1(S1, A1, A2, xl1, xr1, attf1, G, G2, b1, Wl2, bl2, Wr2, br2):
    RB = 1000
    nb = N // RB
    return pl.pallas_call(
        _f1_body,
        grid=(nb,),
        in_specs=[
            pl.BlockSpec((RB, D1 + 16), lambda i: (i, 0)),
            pl.BlockSpec((RB, D1 + 16), lambda i: (i + nb, 0)),
            pl.BlockSpec((RB, D1), lambda i: (i, 0)),
            pl.BlockSpec((RB, D1), lambda i: (i + nb, 0)),
            pl.BlockSpec((RB, 80), lambda i: (i, 0)),
            pl.BlockSpec((RB, 80), lambda i: (i + nb, 0)),
            pl.BlockSpec((RB, D1), lambda i: (i, 0)),
            pl.BlockSpec((RB, D1), lambda i: (i, 0)),
            pl.BlockSpec((1, D1), lambda i: (0, 0)),
            pl.BlockSpec((D1, HEADS), lambda i: (0, 0)),
            pl.BlockSpec((HEADS, D1), lambda i: (0, 0)),
            pl.BlockSpec((1, D1), lambda i: (0, 0)),
            pl.BlockSpec((D1, DOUT), lambda i: (0, 0)),
            pl.BlockSpec((1, DOUT), lambda i: (0, 0)),
            pl.BlockSpec((D1, DOUT), lambda i: (0, 0)),
            pl.BlockSpec((1, DOUT), lambda i: (0, 0)),
        ],
        out_specs=[
            pl.BlockSpec((RB, DOUT), lambda i: (i, 0)),
            pl.BlockSpec((RB, DOUT), lambda i: (i, 0)),
        ],
        out_shape=[
            jax.ShapeDtypeStruct((N, DOUT), _f32),
            jax.ShapeDtypeStruct((N, DOUT), _f32),
        ],
    )(S1, S1, A1, A1, A2, A2, xl1, xr1, attf1, G, G2, b1, Wl2, bl2, Wr2, br2)


def _f2_body(s2a, s2b, a2a, a2b, xl, xr, attf, b2, oo, ol):
    num = s2a[:, 0:DOUT] + s2b[:, 0:DOUT]
    den = s2a[:, DOUT:DOUT + 1] + s2b[:, DOUT:DOUT + 1]
    es2 = a2a[:, 0:DOUT] + a2b[:, 0:DOUT]
    deg = a2a[:, 64:65] + a2b[:, 64:65]
    loop_ee = es2 / jnp.maximum(deg, 1.0)
    xl_ = xl[...]
    t = xl_ + xr[...] + loop_ee
    m = jnp.maximum(t, 0.0) + 0.2 * jnp.minimum(t, 0.0)
    a = jnp.sum(m * attf[...], axis=1, keepdims=True)
    p = jnp.exp(a)
    dent = den + p
    numt = num + xl_ * p
    o = numt / (dent + 1e-16) + b2[...]
    mx = jnp.max(o, axis=1, keepdims=True)
    sh = o - mx
    lse = jnp.log(jnp.sum(jnp.exp(sh), axis=1, keepdims=True))
    oo[...] = o
    ol[...] = sh - lse


def _f2(S2, A2, xl2, xr2, att2f, b2):
    RB = 1000
    nb = N // RB
    return pl.pallas_call(
        _f2_body,
        grid=(nb,),
        in_specs=[
            pl.BlockSpec((RB, 80), lambda i: (i, 0)),
            pl.BlockSpec((RB, 80), lambda i: (i + nb, 0)),
            pl.BlockSpec((RB, 80), lambda i: (i, 0)),
            pl.BlockSpec((RB, 80), lambda i: (i + nb, 0)),
            pl.BlockSpec((RB, DOUT), lambda i: (i, 0)),
            pl.BlockSpec((RB, DOUT), lambda i: (i, 0)),
            pl.BlockSpec((1, DOUT), lambda i: (0, 0)),
            pl.BlockSpec((1, DOUT), lambda i: (0, 0)),
        ],
        out_specs=[
            pl.BlockSpec((RB, DOUT), lambda i: (i, 0)),
            pl.BlockSpec((RB, DOUT), lambda i: (i, 0)),
        ],
        out_shape=[
            jax.ShapeDtypeStruct((N, DOUT), _f32),
            jax.ShapeDtypeStruct((N, DOUT), _f32),
        ],
    )(S2, S2, A2, A2, xl2, xr2, att2f, b2)


_MESH = dict(core_axis_name="c", subcore_axis_name="s", num_cores=NC,
             num_subcores=NS)


BS = 80
NBS = TPE // BS


def _sc_scatter(EW):
    def body(dst_h, ee_h, z_h, o_h, dstv0, dstv1, ev0, ev1, es, semr, semi):
        c = lax.axis_index("c")
        s = lax.axis_index("s")
        wid = c * NS + s
        for k in range(CPS):
            idx = s + k * NS

            @pl.when(idx < NCHUNK)
            def _():
                pltpu.sync_copy(z_h, es.at[pl.ds(idx * CCH, CCH)])
        plsc.subcore_barrier()

        dstv = (dstv0, dstv1)
        ev = (ev0, ev1)

        def rows_at(b):
            return ee_h.at[pl.ds(wid * TPE + b * BS, BS)]

        def idx_at(b):
            return dst_h.at[pl.ds(wid * TPE + b * BS, BS)]

        def phase(b, q, guard):
            o = 1 - q
            pltpu.make_async_copy(rows_at(b), ev[q], semr).wait()
            if guard:
                @pl.when(b + 1 < NBS)
                def _():
                    pltpu.make_async_copy(idx_at(b + 1), dstv[o], semi).wait()
                    pltpu.async_copy(rows_at(b + 1), ev[o], semr)
            pltpu.sync_copy(ev[q], es.at[dstv[q]], add=True)
            if guard:
                @pl.when(b + 2 < NBS)
                def _():
                    pltpu.async_copy(idx_at(b + 2), dstv[q], semi)

        pltpu.sync_copy(idx_at(0), dstv0)
        pltpu.async_copy(rows_at(0), ev0, semr)
        pltpu.async_copy(idx_at(1), dstv1, semi)

        def pair(j, carry):
            phase(2 * j, 0, True)
            phase(2 * j + 1, 1, True)
            return carry

        lax.fori_loop(0, NBS // 2, pair, 0)
        phase(NBS - 1, (NBS - 1) % 2, False)

        plsc.subcore_barrier()
        for k in range(CPS):
            idx = s + k * NS

            @pl.when(idx < NCHUNK)
            def _():
                r0 = idx * CCH
                pltpu.sync_copy(es.at[pl.ds(r0, CCH)],
                                o_h.at[pl.ds(c * N + r0, CCH)])

    return pl.kernel(
        body,
        out_type=jax.ShapeDtypeStruct((NC * N, EW), _f32),
        mesh=plsc.VectorSubcoreMesh(**_MESH),
        scratch_types=[
            pltpu.VMEM((BS,), jnp.int32),
            pltpu.VMEM((BS,), jnp.int32),
            pltpu.VMEM((BS, EW), _f32),
            pltpu.VMEM((BS, EW), _f32),
            pltpu.VMEM_SHARED((N, EW), _f32),
            pltpu.SemaphoreType.DMA,
            pltpu.SemaphoreType.DMA,
        ],
        compiler_params=pltpu.CompilerParams(use_tc_tiling_on_sc=False, needs_layout_passes=False),
    )


def _sc_edge(D, H, dh):
    W = D + 16

    def body(src_h, dst_h, xl_h, xr_h, ee_h, att_h, z_h, S_h,
             srcv0, srcv1, dstv0, dstv1, xlv0, xlv1, xrv0, xrv1,
             eev0, eev1, nxv, attv, acc, semg, semi):
        c = lax.axis_index("c")
        s = lax.axis_index("s")
        wid = c * NS + s
        pltpu.sync_copy(att_h, attv)
        for k in range(CPS):
            idx = s + k * NS

            @pl.when(idx < NCHUNK)
            def _():
                pltpu.sync_copy(z_h, acc.at[pl.ds(idx * CCH, CCH)])
        plsc.subcore_barrier()

        lane = lax.iota(jnp.int32, 16)
        srcv = (srcv0, srcv1)
        dstv = (dstv0, dstv1)
        xlv = (xlv0, xlv1)
        xrv = (xrv0, xrv1)
        eev = (eev0, eev1)

        def edge(e, q):
            den_acc = jnp.full((16,), -1e30, _f32)
            xls = []
            for h in range(H):
                a = jnp.float32(0.0)
                for kk in range(dh // 16):
                    sl = pl.ds(h * dh + kk * 16, 16)
                    xlc = xlv[q][e, sl]
                    xls.append(xlc)
                    t = xlc + xrv[q][e, sl] + eev[q][e, sl]
                    m = jnp.maximum(t, 0.0) + 0.2 * jnp.minimum(t, 0.0)
                    a = a + jnp.sum(m * attv[h, pl.ds(kk * 16, 16)])
                den_acc = jnp.where(lane == h, a, den_acc)
            pb = jnp.exp(den_acc)
            nxv[e, pl.ds(D, 16)] = pb
            for h in range(H):
                p = pb[h]
                for kk in range(dh // 16):
                    sl = pl.ds(h * dh + kk * 16, 16)
                    nxv[e, sl] = p * xls[h * (dh // 16) + kk]

        def seq_at(ref, b):
            return ref.at[pl.ds(wid * TPE + b * B, B)]

        def issue_gathers(b, q):
            pltpu.async_copy(xl_h.at[srcv[q]], xlv[q], semg)
            pltpu.async_copy(xr_h.at[dstv[q]], xrv[q], semg)
            pltpu.async_copy(seq_at(ee_h, b), eev[q], semg)

        def wait_gathers(b, q):
            pltpu.make_async_copy(xl_h.at[srcv[q]], xlv[q], semg).wait()
            pltpu.make_async_copy(xr_h.at[dstv[q]], xrv[q], semg).wait()
            pltpu.make_async_copy(seq_at(ee_h, b), eev[q], semg).wait()

        def phase(b, q, guard):
            o = 1 - q
            wait_gathers(b, q)
            if guard:
                @pl.when(b + 1 < NB)
                def _():
                    pltpu.make_async_copy(seq_at(src_h, b + 1), srcv[o],
                                          semi).wait()
                    pltpu.make_async_copy(seq_at(dst_h, b + 1), dstv[o],
                                          semi).wait()
                    issue_gathers(b + 1, o)
            for e in range(B):
                edge(e, q)
            pltpu.sync_copy(nxv, acc.at[dstv[q]], add=True)
            if guard:
                @pl.when(b + 2 < NB)
                def _():
                    pltpu.async_copy(seq_at(src_h, b + 2), srcv[q], semi)
                    pltpu.async_copy(seq_at(dst_h, b + 2), dstv[q], semi)

        pltpu.sync_copy(seq_at(src_h, 0), srcv0)
        pltpu.sync_copy(seq_at(dst_h, 0), dstv0)
        issue_gathers(0, 0)
        pltpu.async_copy(seq_at(src_h, 1), srcv1, semi)
        pltpu.async_copy(seq_at(dst_h, 1), dstv1, semi)

        def pair(j, carry):
            phase(2 * j, 0, True)
            phase(2 * j + 1, 1, True)
            return carry

        lax.fori_loop(0, NB // 2, pair, 0)

        plsc.subcore_barrier()
        for k in range(CPS):
            idx = s + k * NS

            @pl.when(idx < NCHUNK)
            def _():
                r0 = idx * CCH
                pltpu.sync_copy(acc.at[pl.ds(r0, CCH)],
                                S_h.at[pl.ds(c * N + r0, CCH)])

    EW = 80 if D == DOUT else D
    return pl.kernel(
        body,
        out_type=jax.ShapeDtypeStruct((NC * N, W), _f32),
        mesh=plsc.VectorSubcoreMesh(**_MESH),
        scratch_types=[
            pltpu.VMEM((B,), jnp.int32),
            pltpu.VMEM((B,), jnp.int32),
            pltpu.VMEM((B,), jnp.int32),
            pltpu.VMEM((B,), jnp.int32),
            pltpu.VMEM((B, D), _f32),
            pltpu.VMEM((B, D), _f32),
            pltpu.VMEM((B, D), _f32),
            pltpu.VMEM((B, D), _f32),
            pltpu.VMEM((B, EW), _f32),
            pltpu.VMEM((B, EW), _f32),
            pltpu.VMEM((B, W), _f32),
            pltpu.VMEM((H, dh), _f32),
            pltpu.VMEM_SHARED((N, W), _f32),
            pltpu.SemaphoreType.DMA,
            pltpu.SemaphoreType.DMA,
        ],
        compiler_params=pltpu.CompilerParams(use_tc_tiling_on_sc=False, needs_layout_passes=False),
    )


def kernel(x, edge_index, edge_feats, Wl1, bl1, Wr1, br1, We1, att1, b1,
           Wl2, bl2, Wr2, br2, We2, att2, b2):
    src = edge_index[0]
    dst = edge_index[1]

    ee1, ee2x = _mm_ee(edge_feats, We1, We2)
    xl1, xr1 = _mm_x1(x, Wl1, bl1.reshape(1, D1), Wr1, br1.reshape(1, D1))

    z128 = jnp.zeros((CCH, D1), _f32)
    z80 = jnp.zeros((CCH, 80), _f32)
    z144 = jnp.zeros((CCH, D1 + 16), _f32)

    A1 = _sc_scatter(D1)(dst, ee1, z128)
    A2 = _sc_scatter(80)(dst, ee2x, z80)
    S1 = _sc_edge(D1, HEADS, DH)(src, dst, xl1, xr1, ee1, att1, z144)

    attf1 = att1.reshape(1, D1)
    G = jnp.repeat(jnp.eye(HEADS, dtype=_f32), DH, axis=0)
    G2 = G.T
    xl2, xr2 = _f1(S1, A1, A2, xl1, xr1, attf1, G, G2, b1.reshape(1, D1),
                   Wl2, bl2.reshape(1, DOUT), Wr2, br2.reshape(1, DOUT))

    S2 = _sc_edge(DOUT, 1, DOUT)(src, dst, xl2, xr2, ee2x, att2, z80)
    o, lsm = _f2(S2, A2, xl2, xr2, att2.reshape(1, DOUT), b2.reshape(1, DOUT))
    return o, lsm
